# Initial kernel scaffold; baseline (speedup 1.0000x reference)
import jax
import jax.numpy as jnp
from jax.experimental import pallas as pl


def kernel(x, edge_index, batch, W1_0, b1_0, W2_0, b2_0, W1_1, b1_1, W2_1, b2_1, W1_2, b1_2, W2_2, b2_2, Wh1, bh1, Wh2, bh2):
    raise NotImplementedError("write your pallas kernel here")


# speedup vs baseline: 6.4458x; 6.4458x over previous
import functools

import jax
import jax.numpy as jnp
from jax import lax
from jax.experimental import pallas as pl
from jax.experimental.pallas import tpu as pltpu
from jax.experimental.pallas import tpu_sc as plsc

_NC = 2
_NS = 16
_NW = _NC * _NS
_K = 80
_G = 64
_NEG = -3.0e38

_HI = lax.Precision.HIGHEST


def _make_sc_aggregate(N, D, E):
    EW = E // _NW
    SCC = 25
    SCH = EW // (SCC * _K)
    RPT = N // _NS
    ZCH = RPT // _K
    mesh = plsc.VectorSubcoreMesh(core_axis_name="c", subcore_axis_name="s",
                                  num_cores=_NC, num_subcores=_NS)

    @functools.partial(
        pl.kernel,
        out_type=jax.ShapeDtypeStruct((_NC * N, D), jnp.float32),
        mesh=mesh,
        scratch_types=[
            pltpu.VMEM((SCC, _K), jnp.int32),
            pltpu.VMEM((SCC, _K), jnp.int32),
            pltpu.VMEM((_K, D), jnp.float32),
            pltpu.VMEM_SHARED((N, D), jnp.float32),
            pltpu.SemaphoreType.DMA,
        ],
    )
    def agg(h_hbm, src_hbm, dst_hbm, zeros_hbm, out_hbm,
            sidx, didx, rows, acc, sem):
        c = lax.axis_index("c")
        s = lax.axis_index("s")
        w = c * _NS + s
        r0 = s * RPT

        pltpu.sync_copy(zeros_hbm, rows)
        for kk in range(ZCH):
            pltpu.sync_copy(rows, acc.at[pl.ds(r0 + kk * _K, _K)])
        plsc.subcore_barrier()

        def outer(t, carry):
            pltpu.sync_copy(src_hbm.at[w * SCH + t], sidx)
            pltpu.sync_copy(dst_hbm.at[w * SCH + t], didx)

            def inner(j, c2):
                pltpu.async_copy(h_hbm.at[sidx.at[j]], rows, sem).wait()
                pltpu.sync_copy(rows, acc.at[didx.at[j]], add=True)
                return c2

            lax.fori_loop(0, SCC, inner, 0)
            return carry

        lax.fori_loop(0, SCH, outer, 0)
        plsc.subcore_barrier()

        for kk in range(ZCH):
            pltpu.sync_copy(acc.at[pl.ds(r0 + kk * _K, _K)], rows)
            pltpu.sync_copy(rows, out_hbm.at[pl.ds(c * N + r0 + kk * _K, _K)])

    return agg


def _make_tc_layer(N, D, bn):
    grid = (N // bn,)

    def body(h_ref, a_ref, w1_ref, b1_ref, w2_ref, b2_ref, o_ref):
        z = h_ref[...] + a_ref[0] + a_ref[1]
        z = jnp.maximum(jnp.dot(z, w1_ref[...]) + b1_ref[...], 0.0)
        z = jnp.dot(z, w2_ref[...]) + b2_ref[...]
        o_ref[...] = jnp.maximum(z, 0.0)

    return pl.pallas_call(
        body,
        grid=grid,
        in_specs=[
            pl.BlockSpec((bn, D), lambda i: (i, 0)),
            pl.BlockSpec((2, bn, D), lambda i: (0, i, 0)),
            pl.BlockSpec((D, D), lambda i: (0, 0)),
            pl.BlockSpec((1, D), lambda i: (0, 0)),
            pl.BlockSpec((D, D), lambda i: (0, 0)),
            pl.BlockSpec((1, D), lambda i: (0, 0)),
        ],
        out_specs=pl.BlockSpec((bn, D), lambda i: (i, 0)),
        out_shape=jax.ShapeDtypeStruct((N, D), jnp.float32),
    )


def _make_tc_layer3_pool_head(N, D, bn):
    nb = N // bn
    grid = (nb,)

    def body(h_ref, a_ref, w1_ref, b1_ref, w2_ref, b2_ref,
             bcol_ref, brow_ref, wh1_ref, bh1_ref, wh2r_ref, bh2_ref,
             o_ref, sum_acc, cnt_acc, max_acc):
        pi = pl.program_id(0)

        @pl.when(pi == 0)
        def _():
            sum_acc[...] = jnp.zeros_like(sum_acc)
            cnt_acc[...] = jnp.zeros_like(cnt_acc)
            max_acc[...] = jnp.full_like(max_acc, _NEG)

        z = h_ref[...] + a_ref[0] + a_ref[1]
        z = jnp.maximum(jnp.dot(z, w1_ref[...]) + b1_ref[...], 0.0)
        z = jnp.dot(z, w2_ref[...]) + b2_ref[...]
        h3 = jnp.maximum(z, 0.0)

        brow = brow_ref[0]
        onehot_t = (brow == lax.broadcasted_iota(jnp.int32, (_G, bn), 0))
        onehot_t = onehot_t.astype(jnp.float32)
        sum_acc[...] += jax.lax.dot(onehot_t, h3, precision=_HI)
        cnt_acc[...] += jax.lax.dot(
            onehot_t, jnp.ones((bn, D), jnp.float32), precision=_HI)

        bcol = bcol_ref[...]
        g_lo = brow_ref[0, 0, 0]
        g_hi = brow_ref[0, 0, bn - 1]
        grow = lax.broadcasted_iota(jnp.int32, (_G, 1), 0)

        def mbody(g, carry):
            contrib = jnp.max(jnp.where(bcol == g, h3, _NEG), axis=0,
                              keepdims=True)
            max_acc[...] = jnp.where(
                grow == g, jnp.maximum(max_acc[...], contrib), max_acc[...])
            return carry

        lax.fori_loop(g_lo, g_hi + 1, mbody, 0)

        @pl.when(pi == nb - 1)
        def _():
            cnt = cnt_acc[...]
            mean = sum_acc[...] / jnp.maximum(cnt, 1.0)
            mx = jnp.where(cnt > 0.0, max_acc[...], 0.0)
            gcat = jnp.concatenate([mean, mx], axis=1)
            hid = jnp.maximum(
                jnp.dot(gcat, wh1_ref[...]) + bh1_ref[...], 0.0)
            logits = jnp.sum(hid * wh2r_ref[...], axis=1, keepdims=True)
            logits = logits + bh2_ref[0, 0]
            o_ref[...] = 1.0 / (1.0 + jnp.exp(-logits))

    return pl.pallas_call(
        body,
        grid=grid,
        in_specs=[
            pl.BlockSpec((bn, D), lambda i: (i, 0)),
            pl.BlockSpec((2, bn, D), lambda i: (0, i, 0)),
            pl.BlockSpec((D, D), lambda i: (0, 0)),
            pl.BlockSpec((1, D), lambda i: (0, 0)),
            pl.BlockSpec((D, D), lambda i: (0, 0)),
            pl.BlockSpec((1, D), lambda i: (0, 0)),
            pl.BlockSpec((bn, 1), lambda i: (i, 0)),
            pl.BlockSpec((1, 1, bn), lambda i: (i, 0, 0)),
            pl.BlockSpec((2 * D, D), lambda i: (0, 0)),
            pl.BlockSpec((1, D), lambda i: (0, 0)),
            pl.BlockSpec((1, D), lambda i: (0, 0)),
            pl.BlockSpec((1, 1), lambda i: (0, 0)),
        ],
        out_specs=pl.BlockSpec((_G, 1), lambda i: (0, 0)),
        out_shape=jax.ShapeDtypeStruct((_G, 1), jnp.float32),
        scratch_shapes=[
            pltpu.VMEM((_G, D), jnp.float32),
            pltpu.VMEM((_G, D), jnp.float32),
            pltpu.VMEM((_G, D), jnp.float32),
        ],
    )


def kernel(x, edge_index, batch,
           W1_0, b1_0, W2_0, b2_0,
           W1_1, b1_1, W2_1, b2_1,
           W1_2, b1_2, W2_2, b2_2,
           Wh1, bh1, Wh2, bh2):
    N, D = x.shape
    E = edge_index.shape[1]
    EW = E // _NW
    CH = EW // _K
    Np = -(-N // (_NS * 40)) * (_NS * 40)
    bn = max(b for b in range(8, 1025, 8) if Np % b == 0)

    x = jnp.pad(x, ((0, Np - N), (0, 0)))
    batch_p = jnp.pad(batch, (0, Np - N), constant_values=_G)
    src3 = edge_index[0].reshape(_NW * (CH // 25), 25, _K)
    dst3 = edge_index[1].reshape(_NW * (CH // 25), 25, _K)
    zeros = jnp.zeros((_K, D), jnp.float32)
    bcol = batch_p.reshape(Np, 1)
    brow = batch_p.reshape(Np // bn, 1, bn)

    sc_agg = _make_sc_aggregate(Np, D, E)
    tc_layer = _make_tc_layer(Np, D, bn)
    tc_tail = _make_tc_layer3_pool_head(Np, D, bn)

    b1s = [b1_0.reshape(1, D), b1_1.reshape(1, D), b1_2.reshape(1, D)]
    b2s = [b2_0.reshape(1, D), b2_1.reshape(1, D), b2_2.reshape(1, D)]
    W1s = [W1_0, W1_1, W1_2]
    W2s = [W2_0, W2_1, W2_2]

    h = x
    for layer in range(2):
        a = sc_agg(h, src3, dst3, zeros).reshape(2, Np, D)
        h = tc_layer(h, a, W1s[layer], b1s[layer], W2s[layer], b2s[layer])
    a = sc_agg(h, src3, dst3, zeros).reshape(2, Np, D)
    score = tc_tail(h, a, W1s[2], b1s[2], W2s[2], b2s[2],
                    bcol, brow, Wh1, bh1.reshape(1, D),
                    Wh2.reshape(1, D), bh2.reshape(1, 1))
    return score


# speedup vs baseline: 10.2571x; 1.5913x over previous
import functools

import jax
import jax.numpy as jnp
from jax import lax
from jax.experimental import pallas as pl
from jax.experimental.pallas import tpu as pltpu
from jax.experimental.pallas import tpu_sc as plsc

_NC = 2
_NS = 16
_NW = _NC * _NS
_K = 40
_BUF = 5
_SCC = 50
_G = 64
_NEG = -3.0e38

_HI = lax.Precision.HIGHEST


def _make_sc_aggregate(N, D, E):
    EW = E // _NW
    SCH = EW // (_SCC * _K)
    ROUNDS = _SCC // _BUF
    RPT = N // _NS
    ZCH = RPT // _K
    mesh = plsc.VectorSubcoreMesh(core_axis_name="c", subcore_axis_name="s",
                                  num_cores=_NC, num_subcores=_NS)

    @functools.partial(
        pl.kernel,
        out_type=jax.ShapeDtypeStruct((_NC * N, D), jnp.float32),
        mesh=mesh,
        scratch_types=(
            [pltpu.VMEM((_SCC, _K), jnp.int32)] * 2
            + [pltpu.VMEM((_K, D), jnp.float32)] * _BUF
            + [pltpu.VMEM_SHARED((N, D), jnp.float32)]
            + [pltpu.SemaphoreType.DMA] * (2 * _BUF)
        ),
    )
    def agg(h_hbm, src_hbm, dst_hbm, zeros_hbm, out_hbm,
            sidx, didx, *rest):
        rows = rest[:_BUF]
        acc = rest[_BUF]
        gsem = rest[_BUF + 1:2 * _BUF + 1]
        ssem = rest[2 * _BUF + 1:]
        c = lax.axis_index("c")
        s = lax.axis_index("s")
        w = c * _NS + s
        r0 = s * RPT

        pltpu.sync_copy(zeros_hbm, rows[0])
        for kk in range(ZCH):
            pltpu.async_copy(rows[0], acc.at[pl.ds(r0 + kk * _K, _K)],
                             ssem[0])
        for kk in range(ZCH):
            pltpu.make_async_copy(rows[0], acc.at[pl.ds(r0, _K)],
                                  ssem[0]).wait()
        plsc.subcore_barrier()

        def outer(t, carry):
            pltpu.sync_copy(src_hbm.at[w * SCH + t], sidx)
            pltpu.sync_copy(dst_hbm.at[w * SCH + t], didx)
            for b in range(_BUF):
                pltpu.async_copy(h_hbm.at[sidx.at[b]], rows[b], gsem[b])

            def round_(q, c2):
                for b in range(_BUF):
                    cb = q * _BUF + b
                    pltpu.make_async_copy(h_hbm.at[sidx.at[0]], rows[b],
                                          gsem[b]).wait()
                    pltpu.async_copy(rows[b], acc.at[didx.at[cb]], ssem[b],
                                     add=True)
                for b in range(_BUF):
                    cb = q * _BUF + b
                    pltpu.make_async_copy(rows[b], acc.at[didx.at[0]],
                                          ssem[b]).wait()

                    @pl.when(q < ROUNDS - 1)
                    def _():
                        pltpu.async_copy(h_hbm.at[sidx.at[cb + _BUF]],
                                         rows[b], gsem[b])
                return c2

            lax.fori_loop(0, ROUNDS, round_, 0)
            return carry

        lax.fori_loop(0, SCH, outer, 0)
        plsc.subcore_barrier()

        for kk in range(ZCH):
            b = kk % _BUF
            if kk >= _BUF:
                pltpu.make_async_copy(
                    rows[b], out_hbm.at[pl.ds(c * N + r0, _K)], ssem[b]).wait()
            pltpu.sync_copy(acc.at[pl.ds(r0 + kk * _K, _K)], rows[b])
            pltpu.async_copy(rows[b],
                             out_hbm.at[pl.ds(c * N + r0 + kk * _K, _K)],
                             ssem[b])
        for b in range(_BUF):
            pltpu.make_async_copy(
                rows[b], out_hbm.at[pl.ds(c * N + r0, _K)], ssem[b]).wait()

    return agg


def _make_tc_layer(N, D, bn):
    grid = (N // bn,)

    def body(h_ref, a_ref, w1_ref, b1_ref, w2_ref, b2_ref, o_ref):
        z = h_ref[...] + a_ref[0] + a_ref[1]
        z = jnp.maximum(jnp.dot(z, w1_ref[...]) + b1_ref[...], 0.0)
        z = jnp.dot(z, w2_ref[...]) + b2_ref[...]
        o_ref[...] = jnp.maximum(z, 0.0)

    return pl.pallas_call(
        body,
        grid=grid,
        in_specs=[
            pl.BlockSpec((bn, D), lambda i: (i, 0)),
            pl.BlockSpec((2, bn, D), lambda i: (0, i, 0)),
            pl.BlockSpec((D, D), lambda i: (0, 0)),
            pl.BlockSpec((1, D), lambda i: (0, 0)),
            pl.BlockSpec((D, D), lambda i: (0, 0)),
            pl.BlockSpec((1, D), lambda i: (0, 0)),
        ],
        out_specs=pl.BlockSpec((bn, D), lambda i: (i, 0)),
        out_shape=jax.ShapeDtypeStruct((N, D), jnp.float32),
    )


def _make_tc_layer3_pool_head(N, D, bn):
    nb = N // bn
    grid = (nb,)

    def body(h_ref, a_ref, w1_ref, b1_ref, w2_ref, b2_ref,
             bcol_ref, brow_ref, wh1_ref, bh1_ref, wh2r_ref, bh2_ref,
             o_ref, sum_acc, cnt_acc, max_acc):
        pi = pl.program_id(0)

        @pl.when(pi == 0)
        def _():
            sum_acc[...] = jnp.zeros_like(sum_acc)
            cnt_acc[...] = jnp.zeros_like(cnt_acc)
            max_acc[...] = jnp.full_like(max_acc, _NEG)

        z = h_ref[...] + a_ref[0] + a_ref[1]
        z = jnp.maximum(jnp.dot(z, w1_ref[...]) + b1_ref[...], 0.0)
        z = jnp.dot(z, w2_ref[...]) + b2_ref[...]
        h3 = jnp.maximum(z, 0.0)

        brow = brow_ref[0]
        onehot_t = (brow == lax.broadcasted_iota(jnp.int32, (_G, bn), 0))
        onehot_t = onehot_t.astype(jnp.float32)
        sum_acc[...] += jax.lax.dot(onehot_t, h3, precision=_HI)
        cnt_acc[...] += jax.lax.dot(
            onehot_t, jnp.ones((bn, D), jnp.float32), precision=_HI)

        bcol = bcol_ref[...]
        g_lo = brow_ref[0, 0, 0]
        g_hi = brow_ref[0, 0, bn - 1]
        grow = lax.broadcasted_iota(jnp.int32, (_G, 1), 0)

        def mbody(g, carry):
            contrib = jnp.max(jnp.where(bcol == g, h3, _NEG), axis=0,
                              keepdims=True)
            max_acc[...] = jnp.where(
                grow == g, jnp.maximum(max_acc[...], contrib), max_acc[...])
            return carry

        lax.fori_loop(g_lo, g_hi + 1, mbody, 0)

        @pl.when(pi == nb - 1)
        def _():
            cnt = cnt_acc[...]
            mean = sum_acc[...] / jnp.maximum(cnt, 1.0)
            mx = jnp.where(cnt > 0.0, max_acc[...], 0.0)
            gcat = jnp.concatenate([mean, mx], axis=1)
            hid = jnp.maximum(
                jnp.dot(gcat, wh1_ref[...]) + bh1_ref[...], 0.0)
            logits = jnp.sum(hid * wh2r_ref[...], axis=1, keepdims=True)
            logits = logits + bh2_ref[0, 0]
            o_ref[...] = 1.0 / (1.0 + jnp.exp(-logits))

    return pl.pallas_call(
        body,
        grid=grid,
        in_specs=[
            pl.BlockSpec((bn, D), lambda i: (i, 0)),
            pl.BlockSpec((2, bn, D), lambda i: (0, i, 0)),
            pl.BlockSpec((D, D), lambda i: (0, 0)),
            pl.BlockSpec((1, D), lambda i: (0, 0)),
            pl.BlockSpec((D, D), lambda i: (0, 0)),
            pl.BlockSpec((1, D), lambda i: (0, 0)),
            pl.BlockSpec((bn, 1), lambda i: (i, 0)),
            pl.BlockSpec((1, 1, bn), lambda i: (i, 0, 0)),
            pl.BlockSpec((2 * D, D), lambda i: (0, 0)),
            pl.BlockSpec((1, D), lambda i: (0, 0)),
            pl.BlockSpec((1, D), lambda i: (0, 0)),
            pl.BlockSpec((1, 1), lambda i: (0, 0)),
        ],
        out_specs=pl.BlockSpec((_G, 1), lambda i: (0, 0)),
        out_shape=jax.ShapeDtypeStruct((_G, 1), jnp.float32),
        scratch_shapes=[
            pltpu.VMEM((_G, D), jnp.float32),
            pltpu.VMEM((_G, D), jnp.float32),
            pltpu.VMEM((_G, D), jnp.float32),
        ],
    )


def kernel(x, edge_index, batch,
           W1_0, b1_0, W2_0, b2_0,
           W1_1, b1_1, W2_1, b2_1,
           W1_2, b1_2, W2_2, b2_2,
           Wh1, bh1, Wh2, bh2):
    N, D = x.shape
    E = edge_index.shape[1]
    EW = E // _NW
    CH = EW // _K
    Np = -(-N // (_NS * 40)) * (_NS * 40)
    bn = max(b for b in range(8, 1025, 8) if Np % b == 0)

    x = jnp.pad(x, ((0, Np - N), (0, 0)))
    batch_p = jnp.pad(batch, (0, Np - N), constant_values=_G)
    src3 = edge_index[0].reshape(_NW * (CH // _SCC), _SCC, _K)
    dst3 = edge_index[1].reshape(_NW * (CH // _SCC), _SCC, _K)
    zeros = jnp.zeros((_K, D), jnp.float32)
    bcol = batch_p.reshape(Np, 1)
    brow = batch_p.reshape(Np // bn, 1, bn)

    sc_agg = _make_sc_aggregate(Np, D, E)
    tc_layer = _make_tc_layer(Np, D, bn)
    tc_tail = _make_tc_layer3_pool_head(Np, D, bn)

    b1s = [b1_0.reshape(1, D), b1_1.reshape(1, D), b1_2.reshape(1, D)]
    b2s = [b2_0.reshape(1, D), b2_1.reshape(1, D), b2_2.reshape(1, D)]
    W1s = [W1_0, W1_1, W1_2]
    W2s = [W2_0, W2_1, W2_2]

    h = x
    for layer in range(2):
        a = sc_agg(h, src3, dst3, zeros).reshape(2, Np, D)
        h = tc_layer(h, a, W1s[layer], b1s[layer], W2s[layer], b2s[layer])
    a = sc_agg(h, src3, dst3, zeros).reshape(2, Np, D)
    score = tc_tail(h, a, W1s[2], b1s[2], W2s[2], b2s[2],
                    bcol, brow, Wh1, bh1.reshape(1, D),
                    Wh2.reshape(1, D), bh2.reshape(1, 1))
    return score


# speedup vs baseline: 10.6231x; 1.0357x over previous
import functools

import jax
import jax.numpy as jnp
from jax import lax
from jax.experimental import pallas as pl
from jax.experimental.pallas import tpu as pltpu
from jax.experimental.pallas import tpu_sc as plsc

_NC = 2
_NS = 16
_NW = _NC * _NS
_K = 50
_BUF = 4
_SCC = 40
_ZR = 40
_G = 64
_NEG = -3.0e38

_HI = lax.Precision.HIGHEST


def _make_sc_aggregate(N, D, E):
    EW = E // _NW
    SCH = EW // (_SCC * _K)
    ROUNDS = _SCC // _BUF
    RPT = N // _NS
    ZCH = RPT // _ZR
    mesh = plsc.VectorSubcoreMesh(core_axis_name="c", subcore_axis_name="s",
                                  num_cores=_NC, num_subcores=_NS)

    @functools.partial(
        pl.kernel,
        out_type=jax.ShapeDtypeStruct((_NC * N, D), jnp.float32),
        mesh=mesh,
        scratch_types=(
            [pltpu.VMEM((_SCC, _K), jnp.int32)] * 4
            + [pltpu.VMEM((_K, D), jnp.float32)] * _BUF
            + [pltpu.VMEM_SHARED((N, D), jnp.float32)]
            + [pltpu.SemaphoreType.DMA] * (2 * _BUF + 1)
        ),
    )
    def agg(h_hbm, src_hbm, dst_hbm, zeros_hbm, out_hbm, *rest):
        sidx = rest[0:2]
        didx = rest[2:4]
        rows = rest[4:4 + _BUF]
        acc = rest[4 + _BUF]
        gsem = rest[5 + _BUF:5 + 2 * _BUF]
        ssem = rest[5 + 2 * _BUF:5 + 3 * _BUF]
        psem = rest[5 + 3 * _BUF]
        c = lax.axis_index("c")
        s = lax.axis_index("s")
        w = c * _NS + s
        r0 = s * RPT
        zrows = [r.at[pl.ds(0, _ZR)] for r in rows]

        pltpu.sync_copy(zeros_hbm, zrows[0])
        for kk in range(ZCH):
            pltpu.async_copy(zrows[0], acc.at[pl.ds(r0 + kk * _ZR, _ZR)],
                             ssem[0])
        for kk in range(ZCH):
            pltpu.make_async_copy(zrows[0], acc.at[pl.ds(r0, _ZR)],
                                  ssem[0]).wait()
        plsc.subcore_barrier()

        for t in range(SCH):
            p = t % 2
            if t == 0:
                pltpu.sync_copy(src_hbm.at[w * SCH], sidx[0])
                pltpu.sync_copy(dst_hbm.at[w * SCH], didx[0])
                for b in range(_BUF):
                    pltpu.async_copy(h_hbm.at[sidx[0].at[b]], rows[b],
                                     gsem[b])
            if t + 1 < SCH:
                pltpu.async_copy(src_hbm.at[w * SCH + t + 1], sidx[1 - p],
                                 psem)
                pltpu.async_copy(dst_hbm.at[w * SCH + t + 1], didx[1 - p],
                                 psem)

            def round_(q, c2, p=p, t=t):
                for b in range(_BUF):
                    cb = q * _BUF + b
                    pltpu.make_async_copy(h_hbm.at[sidx[p].at[0]], rows[b],
                                          gsem[b]).wait()
                    pltpu.async_copy(rows[b], acc.at[didx[p].at[cb]],
                                     ssem[b], add=True)
                for b in range(_BUF):
                    cb = q * _BUF + b
                    pltpu.make_async_copy(rows[b], acc.at[didx[p].at[0]],
                                          ssem[b]).wait()

                    @pl.when(q < ROUNDS - 1)
                    def _():
                        pltpu.async_copy(h_hbm.at[sidx[p].at[cb + _BUF]],
                                         rows[b], gsem[b])

                    if t + 1 < SCH:
                        @pl.when(q == ROUNDS - 1)
                        def _():
                            if b == 0:
                                pltpu.make_async_copy(
                                    src_hbm.at[w * SCH], sidx[1 - p],
                                    psem).wait()
                                pltpu.make_async_copy(
                                    dst_hbm.at[w * SCH], didx[1 - p],
                                    psem).wait()
                            pltpu.async_copy(h_hbm.at[sidx[1 - p].at[b]],
                                             rows[b], gsem[b])
                return c2

            lax.fori_loop(0, ROUNDS, round_, 0)
        plsc.subcore_barrier()

        for kk in range(ZCH):
            b = kk % _BUF
            if kk >= _BUF:
                pltpu.make_async_copy(
                    zrows[b], out_hbm.at[pl.ds(c * N + r0, _ZR)],
                    ssem[b]).wait()
            pltpu.sync_copy(acc.at[pl.ds(r0 + kk * _ZR, _ZR)], zrows[b])
            pltpu.async_copy(zrows[b],
                             out_hbm.at[pl.ds(c * N + r0 + kk * _ZR, _ZR)],
                             ssem[b])
        for b in range(min(_BUF, ZCH)):
            pltpu.make_async_copy(
                zrows[b], out_hbm.at[pl.ds(c * N + r0, _ZR)], ssem[b]).wait()

    return agg


def _make_tc_layer(N, D, bn):
    grid = (N // bn,)

    def body(h_ref, a_ref, w1_ref, b1_ref, w2_ref, b2_ref, o_ref):
        z = h_ref[...] + a_ref[0] + a_ref[1]
        z = jnp.maximum(jnp.dot(z, w1_ref[...]) + b1_ref[...], 0.0)
        z = jnp.dot(z, w2_ref[...]) + b2_ref[...]
        o_ref[...] = jnp.maximum(z, 0.0)

    return pl.pallas_call(
        body,
        grid=grid,
        in_specs=[
            pl.BlockSpec((bn, D), lambda i: (i, 0)),
            pl.BlockSpec((2, bn, D), lambda i: (0, i, 0)),
            pl.BlockSpec((D, D), lambda i: (0, 0)),
            pl.BlockSpec((1, D), lambda i: (0, 0)),
            pl.BlockSpec((D, D), lambda i: (0, 0)),
            pl.BlockSpec((1, D), lambda i: (0, 0)),
        ],
        out_specs=pl.BlockSpec((bn, D), lambda i: (i, 0)),
        out_shape=jax.ShapeDtypeStruct((N, D), jnp.float32),
    )


def _make_tc_layer3_pool_head(N, D, bn):
    nb = N // bn
    grid = (nb,)

    def body(h_ref, a_ref, w1_ref, b1_ref, w2_ref, b2_ref,
             bcol_ref, brow_ref, wh1_ref, bh1_ref, wh2r_ref, bh2_ref,
             o_ref, sum_acc, cnt_acc, max_acc):
        pi = pl.program_id(0)

        @pl.when(pi == 0)
        def _():
            sum_acc[...] = jnp.zeros_like(sum_acc)
            cnt_acc[...] = jnp.zeros_like(cnt_acc)
            max_acc[...] = jnp.full_like(max_acc, _NEG)

        z = h_ref[...] + a_ref[0] + a_ref[1]
        z = jnp.maximum(jnp.dot(z, w1_ref[...]) + b1_ref[...], 0.0)
        z = jnp.dot(z, w2_ref[...]) + b2_ref[...]
        h3 = jnp.maximum(z, 0.0)

        brow = brow_ref[0]
        onehot_t = (brow == lax.broadcasted_iota(jnp.int32, (_G, bn), 0))
        onehot_t = onehot_t.astype(jnp.float32)
        sum_acc[...] += jax.lax.dot(onehot_t, h3, precision=_HI)
        cnt_acc[...] += jax.lax.dot(
            onehot_t, jnp.ones((bn, D), jnp.float32), precision=_HI)

        bcol = bcol_ref[...]
        g_lo = brow_ref[0, 0, 0]
        g_hi = brow_ref[0, 0, bn - 1]
        grow = lax.broadcasted_iota(jnp.int32, (_G, 1), 0)

        def mbody(g, carry):
            contrib = jnp.max(jnp.where(bcol == g, h3, _NEG), axis=0,
                              keepdims=True)
            max_acc[...] = jnp.where(
                grow == g, jnp.maximum(max_acc[...], contrib), max_acc[...])
            return carry

        lax.fori_loop(g_lo, g_hi + 1, mbody, 0)

        @pl.when(pi == nb - 1)
        def _():
            cnt = cnt_acc[...]
            mean = sum_acc[...] / jnp.maximum(cnt, 1.0)
            mx = jnp.where(cnt > 0.0, max_acc[...], 0.0)
            gcat = jnp.concatenate([mean, mx], axis=1)
            hid = jnp.maximum(
                jnp.dot(gcat, wh1_ref[...]) + bh1_ref[...], 0.0)
            logits = jnp.sum(hid * wh2r_ref[...], axis=1, keepdims=True)
            logits = logits + bh2_ref[0, 0]
            o_ref[...] = 1.0 / (1.0 + jnp.exp(-logits))

    return pl.pallas_call(
        body,
        grid=grid,
        in_specs=[
            pl.BlockSpec((bn, D), lambda i: (i, 0)),
            pl.BlockSpec((2, bn, D), lambda i: (0, i, 0)),
            pl.BlockSpec((D, D), lambda i: (0, 0)),
            pl.BlockSpec((1, D), lambda i: (0, 0)),
            pl.BlockSpec((D, D), lambda i: (0, 0)),
            pl.BlockSpec((1, D), lambda i: (0, 0)),
            pl.BlockSpec((bn, 1), lambda i: (i, 0)),
            pl.BlockSpec((1, 1, bn), lambda i: (i, 0, 0)),
            pl.BlockSpec((2 * D, D), lambda i: (0, 0)),
            pl.BlockSpec((1, D), lambda i: (0, 0)),
            pl.BlockSpec((1, D), lambda i: (0, 0)),
            pl.BlockSpec((1, 1), lambda i: (0, 0)),
        ],
        out_specs=pl.BlockSpec((_G, 1), lambda i: (0, 0)),
        out_shape=jax.ShapeDtypeStruct((_G, 1), jnp.float32),
        scratch_shapes=[
            pltpu.VMEM((_G, D), jnp.float32),
            pltpu.VMEM((_G, D), jnp.float32),
            pltpu.VMEM((_G, D), jnp.float32),
        ],
    )


def kernel(x, edge_index, batch,
           W1_0, b1_0, W2_0, b2_0,
           W1_1, b1_1, W2_1, b2_1,
           W1_2, b1_2, W2_2, b2_2,
           Wh1, bh1, Wh2, bh2):
    N, D = x.shape
    E = edge_index.shape[1]
    EW = E // _NW
    CH = EW // _K
    Np = -(-N // (_NS * 40)) * (_NS * 40)
    bn = max(b for b in range(8, 1025, 8) if Np % b == 0)

    x = jnp.pad(x, ((0, Np - N), (0, 0)))
    batch_p = jnp.pad(batch, (0, Np - N), constant_values=_G)
    src3 = edge_index[0].reshape(_NW * (CH // _SCC), _SCC, _K)
    dst3 = edge_index[1].reshape(_NW * (CH // _SCC), _SCC, _K)
    zeros = jnp.zeros((_ZR, D), jnp.float32)
    bcol = batch_p.reshape(Np, 1)
    brow = batch_p.reshape(Np // bn, 1, bn)

    sc_agg = _make_sc_aggregate(Np, D, E)
    tc_layer = _make_tc_layer(Np, D, bn)
    tc_tail = _make_tc_layer3_pool_head(Np, D, bn)

    b1s = [b1_0.reshape(1, D), b1_1.reshape(1, D), b1_2.reshape(1, D)]
    b2s = [b2_0.reshape(1, D), b2_1.reshape(1, D), b2_2.reshape(1, D)]
    W1s = [W1_0, W1_1, W1_2]
    W2s = [W2_0, W2_1, W2_2]

    h = x
    for layer in range(2):
        a = sc_agg(h, src3, dst3, zeros).reshape(2, Np, D)
        h = tc_layer(h, a, W1s[layer], b1s[layer], W2s[layer], b2s[layer])
    a = sc_agg(h, src3, dst3, zeros).reshape(2, Np, D)
    score = tc_tail(h, a, W1s[2], b1s[2], W2s[2], b2s[2],
                    bcol, brow, Wh1, bh1.reshape(1, D),
                    Wh2.reshape(1, D), bh2.reshape(1, 1))
    return score
